# Initial kernel scaffold; baseline (speedup 1.0000x reference)
import jax
import jax.numpy as jnp
from jax.experimental import pallas as pl


def kernel(E_u, E_i, edge_weight, edge_index):
    raise NotImplementedError("write your pallas kernel here")


# speedup vs baseline: 4.4890x; 4.4890x over previous
import functools

import jax
import jax.numpy as jnp
from jax import lax
from jax.experimental import pallas as pl
from jax.experimental.pallas import tpu as pltpu
from jax.experimental.pallas import tpu_sc as plsc

N_U = 25000
N_I = 25000
N = N_U + N_I
NP = 51200
D = 64
HALF = 32
NE = 800000
G = 128
NROWS = 6400
NE_PAD = NROWS * G
NC = 2
NS = 16
ROWS_PER_TILE = NROWS // NS
SUP = 16
N_SUP = ROWS_PER_TILE // SUP
NODE_TILE = NP // NS
FCH = 64
N_FCH = NODE_TILE // FCH


def _sc_body(e0h, src2, dst2, w2, outh, acc, eb1, eb2,
             idx_v, dst_v, w_v, rows_v, b0, b1, b2, b3, sem):
    c = lax.axis_index("c")
    s = lax.axis_index("s")
    tbase = s * NODE_TILE
    gbase = c * NP
    ebase = s * ROWS_PER_TILE

    zeros16 = jnp.zeros((16,), jnp.float32)

    def zfill(r, _):
        b0[r, pl.ds(0, 16)] = zeros16
        b0[r, pl.ds(16, 16)] = zeros16
        return _
    lax.fori_loop(0, FCH, zfill, None)

    def do_layer(table, flush):
        def zero_body(k, _):
            pltpu.sync_copy(b0, acc.at[pl.ds(tbase + k * FCH, FCH)])
            return _
        lax.fori_loop(0, N_FCH, zero_body, None)
        plsc.subcore_barrier()

        def sup_body(it, _):
            row0 = ebase + it * SUP
            pltpu.sync_copy(src2.at[c, pl.ds(row0, SUP), :], idx_v)
            pltpu.sync_copy(dst2.at[pl.ds(row0, SUP), :], dst_v)
            pltpu.sync_copy(w2.at[pl.ds(row0, SUP), :], w_v)

            def row_body(j, _):
                pltpu.async_copy(table.at[idx_v.at[j]], rows_v, sem).wait()

                def group_body(g, _):
                    w16 = w_v[j, pl.ds(g * 16, 16)]
                    for lane in range(16):
                        e = g * 16 + lane
                        w = w16[lane]
                        rows_v[e, pl.ds(0, 16)] = rows_v[e, pl.ds(0, 16)] * w
                        rows_v[e, pl.ds(16, 16)] = rows_v[e, pl.ds(16, 16)] * w
                    return _
                lax.fori_loop(0, G // 16, group_body, None)

                pltpu.sync_copy(rows_v, acc.at[dst_v.at[j]], add=True)
                return _
            lax.fori_loop(0, SUP, row_body, None)
            return _
        lax.fori_loop(0, N_SUP, sup_body, None)
        plsc.subcore_barrier()

        if flush is not None:
            def flush_body(k, _):
                r = tbase + k * FCH
                pltpu.sync_copy(acc.at[pl.ds(r, FCH)], b1)
                pltpu.sync_copy(b1, flush.at[pl.ds(gbase + r, FCH)])
                return _
            lax.fori_loop(0, N_FCH, flush_body, None)
            plsc.subcore_barrier()

    do_layer(e0h, eb1)
    do_layer(eb1, eb2)
    do_layer(eb2, None)

    def mean_body(k, _):
        r = tbase + k * FCH
        pltpu.sync_copy(e0h.at[pl.ds(gbase + r, FCH)], b0)
        pltpu.sync_copy(eb1.at[pl.ds(gbase + r, FCH)], b1)
        pltpu.sync_copy(eb2.at[pl.ds(gbase + r, FCH)], b2)
        pltpu.sync_copy(acc.at[pl.ds(r, FCH)], b3)

        def mean_row(rr, _):
            for h in range(2):
                sl = pl.ds(h * 16, 16)
                b0[rr, sl] = (b0[rr, sl] + b1[rr, sl] + b2[rr, sl]
                              + b3[rr, sl]) * 0.25
            return _
        lax.fori_loop(0, FCH, mean_row, None)
        pltpu.sync_copy(b0, outh.at[pl.ds(gbase + r, FCH)])
        return _
    lax.fori_loop(0, N_FCH, mean_body, None)


@jax.jit
def _lightgcn_sc(E0h, src2, dst2, w2):
    mesh = plsc.VectorSubcoreMesh(core_axis_name="c", subcore_axis_name="s")
    f = pl.kernel(
        _sc_body,
        out_type=jax.ShapeDtypeStruct((NC * NP, HALF), jnp.float32),
        mesh=mesh,
        scratch_types=[
            pltpu.VMEM_SHARED((NP, HALF), jnp.float32),
            pltpu.HBM((NC * NP, HALF), jnp.float32),
            pltpu.HBM((NC * NP, HALF), jnp.float32),
            pltpu.VMEM((SUP, G), jnp.int32),
            pltpu.VMEM((SUP, G), jnp.int32),
            pltpu.VMEM((SUP, G), jnp.float32),
            pltpu.VMEM((G, HALF), jnp.float32),
            pltpu.VMEM((FCH, HALF), jnp.float32),
            pltpu.VMEM((FCH, HALF), jnp.float32),
            pltpu.VMEM((FCH, HALF), jnp.float32),
            pltpu.VMEM((FCH, HALF), jnp.float32),
            pltpu.SemaphoreType.DMA,
        ],
        compiler_params=pltpu.CompilerParams(use_tc_tiling_on_sc=False),
    )
    return f(E0h, src2, dst2, w2)


def kernel(E_u, E_i, edge_weight, edge_index):
    E = jnp.concatenate([E_u, E_i], axis=0)
    Eh = E.reshape(N, NC, HALF).transpose(1, 0, 2)
    E0h = jnp.pad(Eh, ((0, 0), (0, NP - N), (0, 0))).reshape(NC * NP, HALF)

    pad = NE_PAD - NE
    src = jnp.pad(edge_index[0], (0, pad))
    dst = jnp.pad(edge_index[1], (0, pad))
    w = jnp.pad(edge_weight, (0, pad))
    src2 = jnp.stack([src, src + NP]).reshape(NC, NROWS, G)
    dst2 = dst.reshape(NROWS, G)
    w2 = w.reshape(NROWS, G)

    outh = _lightgcn_sc(E0h, src2, dst2, w2)
    outh = outh.reshape(NC, NP, HALF)[:, :N, :]
    return outh.transpose(1, 0, 2).reshape(N, D)
